# baseline (speedup 1.0000x reference)
import jax
import jax.numpy as jnp
from jax.experimental import pallas as pl
from jax.experimental.pallas import tpu as pltpu

_B = 256
_K = 28224
_F = 1500

_FT = 768
_KT = 4096
_KB = 7

_IT = 2048
_IB = 14


def _enc_kernel(x_ref, w_ref, be_ref, y_ref, jac_ref, acc_ref, rn2_ref):
    f = pl.program_id(0)
    k = pl.program_id(1)

    @pl.when(k == 0)
    def _init():
        acc_ref[...] = jnp.zeros_like(acc_ref)
        rn2_ref[...] = jnp.zeros_like(rn2_ref)

    lane = jax.lax.broadcasted_iota(jnp.int32, (1, _KT), 1)
    valid = (k * _KT + lane) < _K
    xb = jnp.where(valid, x_ref[...], 0.0)
    wb = jnp.where(valid, w_ref[...], 0.0)

    acc_ref[...] += jax.lax.dot_general(
        xb, wb, (((1,), (1,)), ((), ())),
        preferred_element_type=jnp.float32)
    rn2_ref[...] += jnp.sum(wb * wb, axis=1, keepdims=True)

    @pl.when(k == _KB - 1)
    def _finish():
        y = jax.nn.sigmoid(acc_ref[...] + be_ref[...])
        y_ref[...] = y
        s = y * (1.0 - y)
        s2c = jnp.sum(s * s, axis=0, keepdims=True)
        rn2_row = rn2_ref[...].T
        col = jax.lax.broadcasted_iota(jnp.int32, (1, _FT), 1)
        fvalid = (f * _FT + col) < _F
        val = jnp.sum(jnp.where(fvalid, s2c * rn2_row, 0.0), keepdims=True)
        jac_ref[...] = val.reshape(1, 1, 1)


def _dec_kernel(y_ref, w_ref, bd_ref, o_ref):
    o_ref[...] = jax.nn.sigmoid(
        jax.lax.dot_general(
            y_ref[...], w_ref[...], (((1,), (1,)), ((), ())),
            preferred_element_type=jnp.float32)
        + bd_ref[...])


def kernel(x, W_enc, b_enc, W_dec, b_dec):
    y_enc, jac_parts = pl.pallas_call(
        _enc_kernel,
        grid=(2, _KB),
        in_specs=[
            pl.BlockSpec((_B, _KT), lambda f, k: (0, k)),
            pl.BlockSpec((_FT, _KT), lambda f, k: (f, k)),
            pl.BlockSpec((1, _FT), lambda f, k: (0, f)),
        ],
        out_specs=[
            pl.BlockSpec((_B, _FT), lambda f, k: (0, f)),
            pl.BlockSpec((1, 1, 1), lambda f, k: (f, 0, 0)),
        ],
        out_shape=[
            jax.ShapeDtypeStruct((_B, _F), jnp.float32),
            jax.ShapeDtypeStruct((2, 1, 1), jnp.float32),
        ],
        scratch_shapes=[
            pltpu.VMEM((_B, _FT), jnp.float32),
            pltpu.VMEM((_FT, 1), jnp.float32),
        ],
        compiler_params=pltpu.CompilerParams(
            dimension_semantics=("parallel", "arbitrary")),
    )(x, W_enc, b_enc.reshape(1, _F))

    jac_reg = jnp.sum(jac_parts)

    y_out = pl.pallas_call(
        _dec_kernel,
        grid=(_IB,),
        in_specs=[
            pl.BlockSpec((_B, _F), lambda i: (0, 0)),
            pl.BlockSpec((_IT, _F), lambda i: (i, 0)),
            pl.BlockSpec((1, _IT), lambda i: (0, i)),
        ],
        out_specs=pl.BlockSpec((_B, _IT), lambda i: (0, i)),
        out_shape=jax.ShapeDtypeStruct((_B, _K), jnp.float32),
        compiler_params=pltpu.CompilerParams(
            dimension_semantics=("parallel",)),
    )(y_enc, W_dec, b_dec.reshape(1, _K))

    return y_out, jac_reg
